# Initial kernel scaffold; baseline (speedup 1.0000x reference)
import jax
import jax.numpy as jnp
from jax.experimental import pallas as pl


def kernel(x, edge_index, W1, b1, W2, b2):
    raise NotImplementedError("write your pallas kernel here")


# speedup vs baseline: 25.2381x; 25.2381x over previous
import functools

import jax
import jax.numpy as jnp
from jax import lax
from jax.experimental import pallas as pl
from jax.experimental.pallas import tpu as pltpu
from jax.experimental.pallas import tpu_sc as plsc

NC = 2
NS = 16
NW = NC * NS

CH = 80


def _deg_kernel(e, npad):
    ept = e // NW
    iters = ept // CH
    lnode = npad // NS

    mesh = plsc.VectorSubcoreMesh(core_axis_name="c", subcore_axis_name="s",
                                  num_cores=NC, num_subcores=NS)

    @functools.partial(
        pl.kernel, mesh=mesh,
        out_type=jax.ShapeDtypeStruct((NC * npad,), jnp.float32),
        scratch_types=[
            pltpu.VMEM((CH,), jnp.int32),
            pltpu.VMEM((CH,), jnp.float32),
            pltpu.VMEM((lnode,), jnp.float32),
            pltpu.VMEM_SHARED((npad,), jnp.float32),
        ],
    )
    def k(dst_hbm, zeros_hbm, ones_hbm, deg_out, dstv, onesv, stagev, deg_sp):
        c = lax.axis_index("c")
        s = lax.axis_index("s")
        tile = c * NS + s
        nb = pl.multiple_of(s * lnode, 8)
        pltpu.sync_copy(zeros_hbm.at[pl.ds(nb, lnode)], stagev)
        pltpu.sync_copy(stagev, deg_sp.at[pl.ds(nb, lnode)])
        pltpu.sync_copy(ones_hbm, onesv)
        plsc.subcore_barrier()

        ebase = tile * ept

        def body(i, carry):
            off = pl.multiple_of(ebase + i * CH, 8)
            pltpu.sync_copy(dst_hbm.at[pl.ds(off, CH)], dstv)
            pltpu.sync_copy(onesv, deg_sp.at[dstv], add=True)
            return carry

        lax.fori_loop(0, iters, body, 0)
        plsc.subcore_barrier()
        ob = pl.multiple_of(c * npad + s * lnode, 8)
        pltpu.sync_copy(deg_sp.at[pl.ds(nb, lnode)], stagev)
        pltpu.sync_copy(stagev, deg_out.at[pl.ds(ob, lnode)])

    return k


def _agg_kernel(n, npad, e, d):
    ept = e // NW
    iters = ept // CH
    rows_per_tile = npad // NS

    mesh = plsc.VectorSubcoreMesh(core_axis_name="c", subcore_axis_name="s",
                                  num_cores=NC, num_subcores=NS)

    @functools.partial(
        pl.kernel, mesh=mesh,
        out_type=jax.ShapeDtypeStruct((NC, npad, d), jnp.float32),
        scratch_types=[
            pltpu.VMEM((CH,), jnp.int32),
            pltpu.VMEM((CH,), jnp.int32),
            pltpu.VMEM((CH, d), jnp.float32),
            pltpu.VMEM_SHARED((npad, d), jnp.float32),
            pltpu.SemaphoreType.DMA,
        ],
    )
    def k(y_hbm, src_hbm, dst_hbm, s_out,
          srcv, dstv, rowsv, s_sp, sem):
        c = lax.axis_index("c")
        s = lax.axis_index("s")
        tile = c * NS + s
        rb = s * rows_per_tile

        def zbody(i, carry):
            rowsv[i // (d // 16), pl.ds((i % (d // 16)) * 16, 16)] = (
                jnp.zeros((16,), jnp.float32))
            return carry

        lax.fori_loop(0, CH * (d // 16), zbody, 0)
        nfull = rows_per_tile // CH
        rem = rows_per_tile - nfull * CH
        for j in range(nfull):
            pltpu.sync_copy(rowsv, s_sp.at[pl.ds(rb + j * CH, CH)])
        if rem:
            pltpu.sync_copy(rowsv.at[pl.ds(0, rem)],
                            s_sp.at[pl.ds(rb + nfull * CH, rem)])
        plsc.subcore_barrier()

        ebase = tile * ept

        def body(i, carry):
            off = pl.multiple_of(ebase + i * CH, 8)
            pltpu.sync_copy(src_hbm.at[pl.ds(off, CH)], srcv)
            pltpu.sync_copy(dst_hbm.at[pl.ds(off, CH)], dstv)
            pltpu.async_copy(y_hbm.at[srcv], rowsv, sem).wait()
            pltpu.sync_copy(rowsv, s_sp.at[dstv], add=True)
            return carry

        lax.fori_loop(0, iters, body, 0)
        plsc.subcore_barrier()
        for j in range(nfull):
            sl = pl.ds(rb + j * CH, CH)
            pltpu.sync_copy(s_sp.at[sl], rowsv)
            pltpu.sync_copy(rowsv, s_out.at[c, sl])
        if rem:
            sl = pl.ds(rb + nfull * CH, rem)
            pltpu.sync_copy(s_sp.at[sl], rowsv.at[pl.ds(0, rem)])
            pltpu.sync_copy(rowsv.at[pl.ds(0, rem)], s_out.at[c, sl])

    return k


def _prep_body(deg_ref, x_ref, dinv_ref, y_ref):
    deg = deg_ref[:, 0:1] + deg_ref[:, 1:2] + 1.0
    dinv = lax.rsqrt(deg)
    dinv_ref[...] = dinv
    y_ref[...] = x_ref[...] * dinv


def _mm_body(s_ref, y_ref, dinv_ref, w1_ref, b1_ref, w2_ref, b2_ref,
             mu_ref, ls_ref):
    agg = (s_ref[0] + s_ref[1] + y_ref[...]) * dinv_ref[...]
    mu_ref[...] = (
        jnp.dot(agg, w1_ref[...], preferred_element_type=jnp.float32)
        + b1_ref[...])
    ls_ref[...] = (
        jnp.dot(agg, w2_ref[...], preferred_element_type=jnp.float32)
        + b2_ref[...])


def kernel(x, edge_index, W1, b1, W2, b2):
    n, d = x.shape
    e = edge_index.shape[1]
    oc = W1.shape[1]
    npad = ((n + 8 * NS - 1) // (8 * NS)) * (8 * NS)

    src = edge_index[0]
    dst = edge_index[1]

    zeros_deg = jnp.zeros((npad,), jnp.float32)
    ones_ch = jnp.ones((CH,), jnp.float32)

    deg_flat = _deg_kernel(e, npad)(dst, zeros_deg, ones_ch)
    deg2 = deg_flat.reshape(NC, npad)[:, :n].T

    blk = 1000
    grid = n // blk
    dinv, y = pl.pallas_call(
        _prep_body,
        grid=(grid,),
        in_specs=[
            pl.BlockSpec((blk, 2), lambda i: (i, 0)),
            pl.BlockSpec((blk, d), lambda i: (i, 0)),
        ],
        out_specs=[
            pl.BlockSpec((blk, 1), lambda i: (i, 0)),
            pl.BlockSpec((blk, d), lambda i: (i, 0)),
        ],
        out_shape=[
            jax.ShapeDtypeStruct((n, 1), jnp.float32),
            jax.ShapeDtypeStruct((n, d), jnp.float32),
        ],
    )(deg2, x)

    s_parts = _agg_kernel(n, npad, e, d)(y, src, dst)[:, :n, :]

    b1r = b1.reshape(1, oc)
    b2r = b2.reshape(1, oc)
    mu, logstd = pl.pallas_call(
        _mm_body,
        grid=(grid,),
        in_specs=[
            pl.BlockSpec((NC, blk, d), lambda i: (0, i, 0)),
            pl.BlockSpec((blk, d), lambda i: (i, 0)),
            pl.BlockSpec((blk, 1), lambda i: (i, 0)),
            pl.BlockSpec((d, oc), lambda i: (0, 0)),
            pl.BlockSpec((1, oc), lambda i: (0, 0)),
            pl.BlockSpec((d, oc), lambda i: (0, 0)),
            pl.BlockSpec((1, oc), lambda i: (0, 0)),
        ],
        out_specs=[
            pl.BlockSpec((blk, oc), lambda i: (i, 0)),
            pl.BlockSpec((blk, oc), lambda i: (i, 0)),
        ],
        out_shape=[
            jax.ShapeDtypeStruct((n, oc), jnp.float32),
            jax.ShapeDtypeStruct((n, oc), jnp.float32),
        ],
    )(s_parts, y, dinv, W1, b1r, W2, b2r)

    return (mu, logstd)


# speedup vs baseline: 27.2581x; 1.0800x over previous
import functools

import jax
import jax.numpy as jnp
from jax import lax
from jax.experimental import pallas as pl
from jax.experimental.pallas import tpu as pltpu
from jax.experimental.pallas import tpu_sc as plsc

NC = 2
NS = 16
NW = NC * NS

CH = 128
NB = 4


def _zero_vec16():
    return jnp.zeros((16,), jnp.float32)


def _deg_kernel(ept_pad, npad):
    iters = ept_pad // CH
    lnode = npad // NS

    mesh = plsc.VectorSubcoreMesh(core_axis_name="c", subcore_axis_name="s",
                                  num_cores=NC, num_subcores=NS)

    @functools.partial(
        pl.kernel, mesh=mesh,
        out_type=jax.ShapeDtypeStruct((NC * npad,), jnp.float32),
        scratch_types=[
            pltpu.VMEM((NB, CH), jnp.int32),
            pltpu.VMEM((CH,), jnp.float32),
            pltpu.VMEM((lnode,), jnp.float32),
            pltpu.VMEM_SHARED((npad,), jnp.float32),
            pltpu.SemaphoreType.DMA((NB,)),
        ],
    )
    def k(dst_hbm, deg_out, dstv, onesv, stagev, deg_sp, ssem):
        c = lax.axis_index("c")
        s = lax.axis_index("s")
        tile = c * NS + s

        def fill_ones(i, carry):
            onesv[pl.ds(i * 16, 16)] = jnp.ones((16,), jnp.float32)
            return carry

        lax.fori_loop(0, CH // 16, fill_ones, 0)

        def fill_zero(i, carry):
            stagev[pl.ds(i * 16, 16)] = _zero_vec16()
            return carry

        lax.fori_loop(0, lnode // 16, fill_zero, 0)

        nb = pl.multiple_of(s * lnode, 8)
        pltpu.sync_copy(stagev, deg_sp.at[pl.ds(nb, lnode)])
        plsc.subcore_barrier()

        ebase = tile * ept_pad

        def body(i, carry):
            b = lax.rem(i, NB)

            @pl.when(i >= NB)
            def _():
                pltpu.make_async_copy(onesv, deg_sp.at[dstv.at[b]],
                                      ssem.at[b]).wait()

            off = pl.multiple_of(ebase + i * CH, 8)
            pltpu.sync_copy(dst_hbm.at[pl.ds(off, CH)], dstv.at[b])
            pltpu.async_copy(onesv, deg_sp.at[dstv.at[b]], ssem.at[b],
                             add=True)
            return carry

        lax.fori_loop(0, iters, body, 0)

        def drain(j, carry):
            pltpu.make_async_copy(onesv, deg_sp.at[dstv.at[lax.rem(j, NB)]],
                                  ssem.at[lax.rem(j, NB)]).wait()
            return carry

        lax.fori_loop(0, NB, drain, 0)
        plsc.subcore_barrier()
        ob = pl.multiple_of(c * npad + s * lnode, 8)
        pltpu.sync_copy(deg_sp.at[pl.ds(nb, lnode)], stagev)
        pltpu.sync_copy(stagev, deg_out.at[pl.ds(ob, lnode)])

    return k


def _agg_kernel(npad, ept_pad, d):
    iters = ept_pad // CH
    rows_per_tile = npad // NS
    ncopy = rows_per_tile // CH
    RNB = 2
    G = 1

    mesh = plsc.VectorSubcoreMesh(core_axis_name="c", subcore_axis_name="s",
                                  num_cores=NC, num_subcores=NS)

    @functools.partial(
        pl.kernel, mesh=mesh,
        out_type=jax.ShapeDtypeStruct((NC, npad, d), jnp.float32),
        scratch_types=[
            pltpu.VMEM((RNB, CH), jnp.int32),
            pltpu.VMEM((RNB, CH), jnp.int32),
            pltpu.VMEM((RNB, CH, d), jnp.float32),
            pltpu.VMEM_SHARED((npad, d), jnp.float32),
            pltpu.SemaphoreType.DMA((RNB,)),
            pltpu.SemaphoreType.DMA((RNB,)),
        ],
    )
    def k(y_hbm, src_hbm, dst_hbm, s_out, srcv, dstv, rowsv, s_sp,
          isem, gsem):
        c = lax.axis_index("c")
        s = lax.axis_index("s")
        tile = c * NS + s
        rb = s * rows_per_tile

        def fill_zero(i, carry):
            rowsv[0, i // (d // 16), pl.ds((i % (d // 16)) * 16, 16)] = (
                _zero_vec16())
            return carry

        lax.fori_loop(0, CH * (d // 16), fill_zero, 0)
        for j in range(ncopy):
            pltpu.sync_copy(rowsv.at[0], s_sp.at[pl.ds(rb + j * CH, CH)])
        plsc.subcore_barrier()

        ebase = tile * ept_pad

        def fire_idx(i, b):
            off = pl.multiple_of(ebase + i * CH, 8)
            pltpu.async_copy(src_hbm.at[pl.ds(off, CH)], srcv.at[b],
                             isem.at[b])
            pltpu.async_copy(dst_hbm.at[pl.ds(off, CH)], dstv.at[b],
                             isem.at[b])

        def wait_idx(i, b):
            off = pl.multiple_of(ebase + i * CH, 8)
            pltpu.make_async_copy(src_hbm.at[pl.ds(off, CH)], srcv.at[b],
                                  isem.at[b]).wait()
            pltpu.make_async_copy(dst_hbm.at[pl.ds(off, CH)], dstv.at[b],
                                  isem.at[b]).wait()

        def fire_gather(i, b):
            pltpu.async_copy(y_hbm.at[srcv.at[b]], rowsv.at[b], gsem.at[b])

        for i in range(RNB):
            fire_idx(i, i)
        for i in range(G):
            wait_idx(i, i)
            fire_gather(i, i)

        def body(i, carry):
            b = lax.rem(i, RNB)
            pltpu.make_async_copy(y_hbm.at[srcv.at[b]], rowsv.at[b],
                                  gsem.at[b]).wait()
            pltpu.sync_copy(rowsv.at[b], s_sp.at[dstv.at[b]], add=True)

            @pl.when(i + RNB < iters)
            def _():
                fire_idx(i + RNB, b)

            @pl.when(i + G < iters)
            def _():
                bg = lax.rem(i + G, RNB)
                wait_idx(i + G, bg)
                fire_gather(i + G, bg)

            return carry

        lax.fori_loop(0, iters, body, 0)
        plsc.subcore_barrier()
        for j in range(ncopy):
            sl = pl.ds(rb + j * CH, CH)
            pltpu.sync_copy(s_sp.at[sl], rowsv.at[0])
            pltpu.sync_copy(rowsv.at[0], s_out.at[c, sl])

    return k


def _prep_body(deg_ref, x_ref, dinv_ref, y_ref):
    deg = deg_ref[:, 0:1] + deg_ref[:, 1:2] + 1.0
    dinv = lax.rsqrt(deg)
    dinv_ref[...] = dinv
    y_ref[...] = x_ref[...] * dinv


def _mm_body(s_ref, y_ref, dinv_ref, w1_ref, b1_ref, w2_ref, b2_ref,
             mu_ref, ls_ref):
    agg = (s_ref[0] + s_ref[1] + y_ref[...]) * dinv_ref[...]
    mu_ref[...] = (
        jnp.dot(agg, w1_ref[...], preferred_element_type=jnp.float32)
        + b1_ref[...])
    ls_ref[...] = (
        jnp.dot(agg, w2_ref[...], preferred_element_type=jnp.float32)
        + b2_ref[...])


def kernel(x, edge_index, W1, b1, W2, b2):
    n, d = x.shape
    e = edge_index.shape[1]
    oc = W1.shape[1]
    npad = ((n + NS * 16 - 1) // (NS * 16)) * (NS * 16)
    npad_agg = ((n + NS * CH - 1) // (NS * CH)) * (NS * CH)

    ept = e // NW
    iters = (ept + CH - 1) // CH
    ept_pad = iters * CH
    pad = ept_pad - ept
    src2 = edge_index[0].reshape(NW, ept)
    dst2 = edge_index[1].reshape(NW, ept)
    if pad:
        src2 = jnp.concatenate(
            [src2, jnp.zeros((NW, pad), jnp.int32)], axis=1)
        dst2 = jnp.concatenate(
            [dst2, jnp.full((NW, pad), n, jnp.int32)], axis=1)
    srcf = src2.reshape(NW * ept_pad)
    dstf = dst2.reshape(NW * ept_pad)

    deg_flat = _deg_kernel(ept_pad, npad)(dstf)
    deg2 = deg_flat.reshape(NC, npad)[:, :n].T

    blk = 1000
    grid = n // blk
    dinv, y = pl.pallas_call(
        _prep_body,
        grid=(grid,),
        in_specs=[
            pl.BlockSpec((blk, 2), lambda i: (i, 0)),
            pl.BlockSpec((blk, d), lambda i: (i, 0)),
        ],
        out_specs=[
            pl.BlockSpec((blk, 1), lambda i: (i, 0)),
            pl.BlockSpec((blk, d), lambda i: (i, 0)),
        ],
        out_shape=[
            jax.ShapeDtypeStruct((n, 1), jnp.float32),
            jax.ShapeDtypeStruct((n, d), jnp.float32),
        ],
    )(deg2, x)

    s_parts = _agg_kernel(npad_agg, ept_pad, d)(y, srcf, dstf)[:, :n, :]

    b1r = b1.reshape(1, oc)
    b2r = b2.reshape(1, oc)
    mu, logstd = pl.pallas_call(
        _mm_body,
        grid=(grid,),
        in_specs=[
            pl.BlockSpec((NC, blk, d), lambda i: (0, i, 0)),
            pl.BlockSpec((blk, d), lambda i: (i, 0)),
            pl.BlockSpec((blk, 1), lambda i: (i, 0)),
            pl.BlockSpec((d, oc), lambda i: (0, 0)),
            pl.BlockSpec((1, oc), lambda i: (0, 0)),
            pl.BlockSpec((d, oc), lambda i: (0, 0)),
            pl.BlockSpec((1, oc), lambda i: (0, 0)),
        ],
        out_specs=[
            pl.BlockSpec((blk, oc), lambda i: (i, 0)),
            pl.BlockSpec((blk, oc), lambda i: (i, 0)),
        ],
        out_shape=[
            jax.ShapeDtypeStruct((n, oc), jnp.float32),
            jax.ShapeDtypeStruct((n, oc), jnp.float32),
        ],
    )(s_parts, y, dinv, W1, b1r, W2, b2r)

    return (mu, logstd)


# speedup vs baseline: 47.6493x; 1.7481x over previous
import functools

import jax
import jax.numpy as jnp
from jax import lax
from jax.experimental import pallas as pl
from jax.experimental.pallas import tpu as pltpu
from jax.experimental.pallas import tpu_sc as plsc

NC = 2
NS = 16
NW = NC * NS

CH = 128
CHA = 80
NB = 4


def _zero_vec16():
    return jnp.zeros((16,), jnp.float32)


def _deg_kernel(ept_pad, npad):
    iters = ept_pad // CH
    lnode = npad // NS

    mesh = plsc.VectorSubcoreMesh(core_axis_name="c", subcore_axis_name="s",
                                  num_cores=NC, num_subcores=NS)

    @functools.partial(
        pl.kernel, mesh=mesh,
        out_type=jax.ShapeDtypeStruct((NC * npad,), jnp.float32),
        scratch_types=[
            pltpu.VMEM((NB, CH), jnp.int32),
            pltpu.VMEM((CH,), jnp.float32),
            pltpu.VMEM((lnode,), jnp.float32),
            pltpu.VMEM_SHARED((npad,), jnp.float32),
            pltpu.SemaphoreType.DMA((NB,)),
        ],
    )
    def k(dst_hbm, deg_out, dstv, onesv, stagev, deg_sp, ssem):
        c = lax.axis_index("c")
        s = lax.axis_index("s")
        tile = c * NS + s

        def fill_ones(i, carry):
            onesv[pl.ds(i * 16, 16)] = jnp.ones((16,), jnp.float32)
            return carry

        lax.fori_loop(0, CH // 16, fill_ones, 0)

        def fill_zero(i, carry):
            stagev[pl.ds(i * 16, 16)] = _zero_vec16()
            return carry

        lax.fori_loop(0, lnode // 16, fill_zero, 0)

        nb = pl.multiple_of(s * lnode, 8)
        pltpu.sync_copy(stagev, deg_sp.at[pl.ds(nb, lnode)])
        plsc.subcore_barrier()

        ebase = tile * ept_pad

        def body(i, carry):
            b = lax.rem(i, NB)

            @pl.when(i >= NB)
            def _():
                pltpu.make_async_copy(onesv, deg_sp.at[dstv.at[b]],
                                      ssem.at[b]).wait()

            off = pl.multiple_of(ebase + i * CH, 8)
            pltpu.sync_copy(dst_hbm.at[pl.ds(off, CH)], dstv.at[b])
            pltpu.async_copy(onesv, deg_sp.at[dstv.at[b]], ssem.at[b],
                             add=True)
            return carry

        lax.fori_loop(0, iters, body, 0)

        def drain(j, carry):
            pltpu.make_async_copy(onesv, deg_sp.at[dstv.at[lax.rem(j, NB)]],
                                  ssem.at[lax.rem(j, NB)]).wait()
            return carry

        lax.fori_loop(0, NB, drain, 0)
        plsc.subcore_barrier()
        ob = pl.multiple_of(c * npad + s * lnode, 8)
        pltpu.sync_copy(deg_sp.at[pl.ds(nb, lnode)], stagev)
        pltpu.sync_copy(stagev, deg_out.at[pl.ds(ob, lnode)])

    return k


def _agg_kernel(npad, ept_pad, d):
    iters = ept_pad // CHA
    rows_per_tile = npad // NS
    ncopy = rows_per_tile // CHA
    RNB = 4
    G = 2

    mesh = plsc.VectorSubcoreMesh(core_axis_name="c", subcore_axis_name="s",
                                  num_cores=NC, num_subcores=NS)

    @functools.partial(
        pl.kernel, mesh=mesh,
        out_type=jax.ShapeDtypeStruct((NC, npad, d), jnp.float32),
        scratch_types=[
            pltpu.VMEM((RNB, CHA), jnp.int32),
            pltpu.VMEM((RNB, CHA), jnp.int32),
            pltpu.VMEM((RNB, CHA, d), jnp.float32),
            pltpu.VMEM_SHARED((npad, d), jnp.float32),
            pltpu.SemaphoreType.DMA((RNB,)),
            pltpu.SemaphoreType.DMA((RNB,)),
        ],
    )
    def k(y_hbm, src_hbm, dst_hbm, s_out, srcv, dstv, rowsv, s_sp,
          isem, gsem):
        c = lax.axis_index("c")
        s = lax.axis_index("s")
        tile = c * NS + s
        rb = s * rows_per_tile

        def fill_zero(i, carry):
            rowsv[0, i // (d // 16), pl.ds((i % (d // 16)) * 16, 16)] = (
                _zero_vec16())
            return carry

        lax.fori_loop(0, CHA * (d // 16), fill_zero, 0)
        for j in range(ncopy):
            pltpu.sync_copy(rowsv.at[0], s_sp.at[pl.ds(rb + j * CHA, CHA)])
        plsc.subcore_barrier()

        ebase = tile * ept_pad

        def fire_idx(i, b):
            off = pl.multiple_of(ebase + i * CHA, 8)
            pltpu.async_copy(src_hbm.at[pl.ds(off, CHA)], srcv.at[b],
                             isem.at[b])
            pltpu.async_copy(dst_hbm.at[pl.ds(off, CHA)], dstv.at[b],
                             isem.at[b])

        def wait_idx(i, b):
            off = pl.multiple_of(ebase + i * CHA, 8)
            pltpu.make_async_copy(src_hbm.at[pl.ds(off, CHA)], srcv.at[b],
                                  isem.at[b]).wait()
            pltpu.make_async_copy(dst_hbm.at[pl.ds(off, CHA)], dstv.at[b],
                                  isem.at[b]).wait()

        def fire_gather(i, b):
            pltpu.async_copy(y_hbm.at[srcv.at[b]], rowsv.at[b], gsem.at[b])

        for i in range(RNB):
            fire_idx(i, i)
        for i in range(G):
            wait_idx(i, i)
            fire_gather(i, i)

        def body(i, carry):
            b = lax.rem(i, RNB)
            pltpu.make_async_copy(y_hbm.at[srcv.at[b]], rowsv.at[b],
                                  gsem.at[b]).wait()
            pltpu.sync_copy(rowsv.at[b], s_sp.at[dstv.at[b]], add=True)

            @pl.when(i + RNB < iters)
            def _():
                fire_idx(i + RNB, b)

            @pl.when(i + G < iters)
            def _():
                bg = lax.rem(i + G, RNB)
                wait_idx(i + G, bg)
                fire_gather(i + G, bg)

            return carry

        lax.fori_loop(0, iters, body, 0)
        plsc.subcore_barrier()
        for j in range(ncopy):
            sl = pl.ds(rb + j * CHA, CHA)
            pltpu.sync_copy(s_sp.at[sl], rowsv.at[0])
            pltpu.sync_copy(rowsv.at[0], s_out.at[c, sl])

    return k


def _prep_body(deg_ref, x_ref, dinv_ref, y_ref):
    deg = deg_ref[:, 0:1] + deg_ref[:, 1:2] + 1.0
    dinv = lax.rsqrt(deg)
    dinv_ref[...] = dinv
    y_ref[...] = x_ref[...] * dinv


def _mm_body(s_ref, y_ref, dinv_ref, w1_ref, b1_ref, w2_ref, b2_ref,
             mu_ref, ls_ref):
    agg = (s_ref[0] + s_ref[1] + y_ref[...]) * dinv_ref[...]
    mu_ref[...] = (
        jnp.dot(agg, w1_ref[...], preferred_element_type=jnp.float32)
        + b1_ref[...])
    ls_ref[...] = (
        jnp.dot(agg, w2_ref[...], preferred_element_type=jnp.float32)
        + b2_ref[...])


def kernel(x, edge_index, W1, b1, W2, b2):
    n, d = x.shape
    e = edge_index.shape[1]
    oc = W1.shape[1]
    npad = ((n + NS * 16 - 1) // (NS * 16)) * (NS * 16)
    npad_agg = ((n + NS * CH - 1) // (NS * CH)) * (NS * CH)

    ept = e // NW
    iters = (ept + CHA - 1) // CHA
    ept_pad = iters * CHA
    pad = ept_pad - ept
    src2 = edge_index[0].reshape(NW, ept)
    dst2 = edge_index[1].reshape(NW, ept)
    if pad:
        src2 = jnp.concatenate(
            [src2, jnp.zeros((NW, pad), jnp.int32)], axis=1)
        dst2 = jnp.concatenate(
            [dst2, jnp.full((NW, pad), n, jnp.int32)], axis=1)
    srcf = src2.reshape(NW * ept_pad)
    dstf = dst2.reshape(NW * ept_pad)

    deg_flat = _deg_kernel(ept_pad, npad)(dstf)
    deg2 = deg_flat.reshape(NC, npad)[:, :n].T

    blk = 1000
    grid = n // blk
    dinv, y = pl.pallas_call(
        _prep_body,
        grid=(grid,),
        in_specs=[
            pl.BlockSpec((blk, 2), lambda i: (i, 0)),
            pl.BlockSpec((blk, d), lambda i: (i, 0)),
        ],
        out_specs=[
            pl.BlockSpec((blk, 1), lambda i: (i, 0)),
            pl.BlockSpec((blk, d), lambda i: (i, 0)),
        ],
        out_shape=[
            jax.ShapeDtypeStruct((n, 1), jnp.float32),
            jax.ShapeDtypeStruct((n, d), jnp.float32),
        ],
    )(deg2, x)

    s_parts = _agg_kernel(npad_agg, ept_pad, d)(y, srcf, dstf)[:, :n, :]

    b1r = b1.reshape(1, oc)
    b2r = b2.reshape(1, oc)
    mu, logstd = pl.pallas_call(
        _mm_body,
        grid=(grid,),
        in_specs=[
            pl.BlockSpec((NC, blk, d), lambda i: (0, i, 0)),
            pl.BlockSpec((blk, d), lambda i: (i, 0)),
            pl.BlockSpec((blk, 1), lambda i: (i, 0)),
            pl.BlockSpec((d, oc), lambda i: (0, 0)),
            pl.BlockSpec((1, oc), lambda i: (0, 0)),
            pl.BlockSpec((d, oc), lambda i: (0, 0)),
            pl.BlockSpec((1, oc), lambda i: (0, 0)),
        ],
        out_specs=[
            pl.BlockSpec((blk, oc), lambda i: (i, 0)),
            pl.BlockSpec((blk, oc), lambda i: (i, 0)),
        ],
        out_shape=[
            jax.ShapeDtypeStruct((n, oc), jnp.float32),
            jax.ShapeDtypeStruct((n, oc), jnp.float32),
        ],
    )(s_parts, y, dinv, W1, b1r, W2, b2r)

    return (mu, logstd)
